# Initial kernel scaffold; baseline (speedup 1.0000x reference)
import jax
import jax.numpy as jnp
from jax.experimental import pallas as pl


def kernel(x, edge_index, edge_attr, W, att_src, att_dst, W_edge, att_edge, b_conv, W_fc, b_fc, ln_g, ln_b, W_gate, b_gate, W_glob, b_glob):
    raise NotImplementedError("write your pallas kernel here")


# speedup vs baseline: 74.5684x; 74.5684x over previous
import functools

import jax
import jax.numpy as jnp
from jax import lax
from jax.experimental import pallas as pl
from jax.experimental.pallas import tpu as pltpu
from jax.experimental.pallas import tpu_sc as plsc

N = 10000
E = 320000
D = 128
H = 8
HD = 16
DE = 4

NC = 2
NS = 16
NW = NC * NS
EB = 128
NBLK = E // EB
STEPS = (NBLK + NW - 1) // NW
NPAD = 10240
ROWS_PER_TILE = NPAD // NS


def _node_tables_body(x_ref, wall_ref, asrc_ref, adst_ref, h_ref, as_ref, ad_ref):
    h = jnp.dot(x_ref[...], wall_ref[...], preferred_element_type=jnp.float32)
    h_ref[...] = h
    as_ref[...] = jnp.dot(h, asrc_ref[...], preferred_element_type=jnp.float32)
    ad_ref[...] = jnp.dot(h, adst_ref[...], preferred_element_type=jnp.float32)


def _node_tables(x, w_all, a_src16, a_dst16):
    bn = 1024
    return pl.pallas_call(
        _node_tables_body,
        grid=(NPAD // bn,),
        in_specs=[
            pl.BlockSpec((bn, D), lambda i: (i, 0)),
            pl.BlockSpec((D, D), lambda i: (0, 0)),
            pl.BlockSpec((D, 16), lambda i: (0, 0)),
            pl.BlockSpec((D, 16), lambda i: (0, 0)),
        ],
        out_specs=[
            pl.BlockSpec((bn, D), lambda i: (i, 0)),
            pl.BlockSpec((bn, 16), lambda i: (i, 0)),
            pl.BlockSpec((bn, 16), lambda i: (i, 0)),
        ],
        out_shape=[
            jax.ShapeDtypeStruct((NPAD, D), jnp.float32),
            jax.ShapeDtypeStruct((NPAD, 16), jnp.float32),
            jax.ShapeDtypeStruct((NPAD, 16), jnp.float32),
        ],
    )(x, w_all, a_src16, a_dst16)


def _edge_table_body(ea_ref, wc_ref, ae_ref):
    ae_ref[...] = jnp.dot(ea_ref[...], wc_ref[...], preferred_element_type=jnp.float32)


def _edge_table(edge_attr, wc16):
    be = 4000
    return pl.pallas_call(
        _edge_table_body,
        grid=(E // be,),
        in_specs=[
            pl.BlockSpec((be, DE), lambda i: (i, 0)),
            pl.BlockSpec((DE, 16), lambda i: (0, 0)),
        ],
        out_specs=pl.BlockSpec((be, 16), lambda i: (i, 0)),
        out_shape=jax.ShapeDtypeStruct((E, 16), jnp.float32),
    )(edge_attr, wc16)


def _edge_kernel_body(src_hbm, dst_hbm, h_hbm, as_hbm, ad_hbm, ae_hbm,
                      znum_hbm, zden_hbm,
                      num0_hbm, num1_hbm, den0_hbm, den1_hbm,
                      src_idx, dst_idx, as_b, ad_b, ae_b, h_b, w_b,
                      num_sh, den_sh, sem0, sem1, sem2):
    c = lax.axis_index("c")
    s = lax.axis_index("s")
    wid = s * NC + c

    r0 = s * ROWS_PER_TILE
    pltpu.sync_copy(znum_hbm.at[pl.ds(r0, ROWS_PER_TILE)],
                    num_sh.at[pl.ds(r0, ROWS_PER_TILE)])
    pltpu.sync_copy(zden_hbm.at[pl.ds(r0, ROWS_PER_TILE)],
                    den_sh.at[pl.ds(r0, ROWS_PER_TILE)])
    plsc.subcore_barrier()

    lanes = lax.iota(jnp.int32, 16)
    headmask = lanes < H

    def step(i, carry):
        b = wid + i * NW

        @pl.when(b < NBLK)
        def _():
            base = b * EB
            pltpu.sync_copy(src_hbm.at[pl.ds(base, EB)], src_idx)
            pltpu.sync_copy(dst_hbm.at[pl.ds(base, EB)], dst_idx)
            pltpu.sync_copy(ae_hbm.at[pl.ds(base, EB)], ae_b)
            pltpu.async_copy(as_hbm.at[src_idx], as_b, sem0).wait()
            pltpu.async_copy(ad_hbm.at[dst_idx], ad_b, sem1).wait()
            pltpu.async_copy(h_hbm.at[src_idx], h_b, sem2).wait()

            def edge_body(j, carry2):
                u = as_b[j, :] + ad_b[j, :] + ae_b[j, :]
                u = jnp.where(u >= 0.0, u, 0.2 * u)
                w = jnp.exp(u)
                w = jnp.where(headmask, w, 0.0)
                w_b[j, :] = w
                for k in range(H):
                    h_b[j, pl.ds(k * HD, HD)] = h_b[j, pl.ds(k * HD, HD)] * w[k]
                return carry2

            lax.fori_loop(0, EB, edge_body, 0)

            pltpu.sync_copy(h_b, num_sh.at[dst_idx], add=True)
            pltpu.sync_copy(w_b, den_sh.at[dst_idx], add=True)

        return carry

    lax.fori_loop(0, STEPS, step, 0)
    plsc.subcore_barrier()

    @pl.when(c == 0)
    def _():
        pltpu.sync_copy(num_sh.at[pl.ds(r0, ROWS_PER_TILE)],
                        num0_hbm.at[pl.ds(r0, ROWS_PER_TILE)])
        pltpu.sync_copy(den_sh.at[pl.ds(r0, ROWS_PER_TILE)],
                        den0_hbm.at[pl.ds(r0, ROWS_PER_TILE)])

    @pl.when(c == 1)
    def _():
        pltpu.sync_copy(num_sh.at[pl.ds(r0, ROWS_PER_TILE)],
                        num1_hbm.at[pl.ds(r0, ROWS_PER_TILE)])
        pltpu.sync_copy(den_sh.at[pl.ds(r0, ROWS_PER_TILE)],
                        den1_hbm.at[pl.ds(r0, ROWS_PER_TILE)])


def _edge_phase(src, dst, h_all, as16, ad16, ae16):
    znum = jnp.zeros((NPAD, D), jnp.float32)
    zden = jnp.zeros((NPAD, 16), jnp.float32)
    run = functools.partial(
        pl.kernel,
        out_type=[
            jax.ShapeDtypeStruct((NPAD, D), jnp.float32),
            jax.ShapeDtypeStruct((NPAD, D), jnp.float32),
            jax.ShapeDtypeStruct((NPAD, 16), jnp.float32),
            jax.ShapeDtypeStruct((NPAD, 16), jnp.float32),
        ],
        mesh=plsc.VectorSubcoreMesh(core_axis_name="c", subcore_axis_name="s"),
        compiler_params=pltpu.CompilerParams(use_tc_tiling_on_sc=False),
        scratch_types=[
            pltpu.VMEM((EB,), jnp.int32),
            pltpu.VMEM((EB,), jnp.int32),
            pltpu.VMEM((EB, 16), jnp.float32),
            pltpu.VMEM((EB, 16), jnp.float32),
            pltpu.VMEM((EB, 16), jnp.float32),
            pltpu.VMEM((EB, D), jnp.float32),
            pltpu.VMEM((EB, 16), jnp.float32),
            pltpu.VMEM_SHARED((NPAD, D), jnp.float32),
            pltpu.VMEM_SHARED((NPAD, 16), jnp.float32),
            pltpu.SemaphoreType.DMA,
            pltpu.SemaphoreType.DMA,
            pltpu.SemaphoreType.DMA,
        ],
    )(_edge_kernel_body)
    return run(src, dst, h_all, as16, ad16, ae16, znum, zden)


def _leaky(v, s):
    return jnp.where(v >= 0.0, v, s * v)


def _epilogue_body(num0_ref, num1_ref, den0_ref, den1_ref, rep_ref,
                   bconv_ref, wfc_ref, bfc_ref, lng_ref, lnb_ref,
                   wgate_ref, bgate_ref, wglob_ref, bglob_ref, out_ref):
    den = den0_ref[pl.ds(0, N), :] + den1_ref[pl.ds(0, N), :]
    den_rep = jnp.dot(den, rep_ref[...], preferred_element_type=jnp.float32)
    x = ((num0_ref[pl.ds(0, N), :] + num1_ref[pl.ds(0, N), :])
         / (den_rep + 1e-16) + bconv_ref[...])
    t = jnp.dot(x, wfc_ref[...], preferred_element_type=jnp.float32) + bfc_ref[...]
    t = _leaky(t, 0.01)
    t = t - jnp.max(t, axis=-1, keepdims=True)
    et = jnp.exp(t)
    sa = et / jnp.sum(et, axis=-1, keepdims=True)
    x = _leaky(x * sa, 0.2)
    x = jnp.dot(x, wfc_ref[...], preferred_element_type=jnp.float32) + bfc_ref[...]
    mu = jnp.mean(x, axis=-1, keepdims=True)
    xc = x - mu
    var = jnp.mean(xc * xc, axis=-1, keepdims=True)
    x = xc * jax.lax.rsqrt(var + 1e-5) * lng_ref[...] + lnb_ref[...]
    nrm = jnp.sqrt(jnp.sum(x * x, axis=-1, keepdims=True))
    x = x / jnp.maximum(nrm, 1e-12)
    g = jnp.sum(x * wgate_ref[...], axis=-1, keepdims=True) + bgate_ref[0, 0]
    g = g - jnp.max(g)
    eg = jnp.exp(g)
    gate = eg / jnp.sum(eg)
    xg = jnp.sum(gate * x, axis=0, keepdims=True)
    q = jnp.dot(xg, wglob_ref[...], preferred_element_type=jnp.float32) + bglob_ref[...]
    q = jnp.maximum(q, 0.0)
    q = q - jnp.max(q, axis=-1, keepdims=True)
    eq = jnp.exp(q)
    ga = eq / jnp.sum(eq, axis=-1, keepdims=True)
    out_ref[...] = x * ga


def _epilogue(num0, num1, den0, den1, rep, bconv_row, wfc, bfc_row, lng_row,
              lnb_row, wgate_row, bgate2, wglob, bglob_row):
    return pl.pallas_call(
        _epilogue_body,
        out_shape=jax.ShapeDtypeStruct((N, D), jnp.float32),
    )(num0, num1, den0, den1, rep, bconv_row, wfc, bfc_row, lng_row,
      lnb_row, wgate_row, bgate2, wglob, bglob_row)


def kernel(x, edge_index, edge_attr, W, att_src, att_dst, W_edge, att_edge,
           b_conv, W_fc, b_fc, ln_g, ln_b, W_gate, b_gate, W_glob, b_glob):
    src = edge_index[0].astype(jnp.int32)
    dst = edge_index[1].astype(jnp.int32)

    w_all = jnp.transpose(W, (1, 0, 2)).reshape(D, H * HD)
    eye = jnp.eye(H, dtype=jnp.float32)
    a_src16 = jnp.concatenate(
        [(eye[:, None, :] * att_src[:, :, None]).reshape(H * HD, H),
         jnp.zeros((H * HD, 16 - H), jnp.float32)], axis=1)
    a_dst16 = jnp.concatenate(
        [(eye[:, None, :] * att_dst[:, :, None]).reshape(H * HD, H),
         jnp.zeros((H * HD, 16 - H), jnp.float32)], axis=1)
    wc16 = jnp.concatenate(
        [jnp.einsum("hdk,hk->dh", W_edge, att_edge),
         jnp.zeros((DE, 16 - H), jnp.float32)], axis=1)
    rep = jnp.concatenate(
        [jnp.repeat(jnp.eye(H, dtype=jnp.float32), HD, axis=1),
         jnp.zeros((16 - H, D), jnp.float32)], axis=0)

    xp = jnp.concatenate([x, jnp.zeros((NPAD - N, D), jnp.float32)], axis=0)
    h_all, as16, ad16 = _node_tables(xp, w_all, a_src16, a_dst16)
    ae16 = _edge_table(edge_attr, wc16)
    num0, num1, den0, den1 = _edge_phase(src, dst, h_all, as16, ad16, ae16)

    return _epilogue(
        num0, num1, den0, den1, rep,
        b_conv.reshape(1, D), W_fc, b_fc.reshape(1, D), ln_g.reshape(1, D),
        ln_b.reshape(1, D), W_gate.reshape(1, D), b_gate.reshape(1, 1),
        W_glob, b_glob.reshape(1, D))


# speedup vs baseline: 89.5394x; 1.2008x over previous
import functools

import jax
import jax.numpy as jnp
from jax import lax
from jax.experimental import pallas as pl
from jax.experimental.pallas import tpu as pltpu
from jax.experimental.pallas import tpu_sc as plsc

N = 10000
E = 320000
D = 128
H = 8
HD = 16
DE = 4

NC = 2
NS = 16
NW = NC * NS
EB = 128
NBLK = E // EB
STEPS = (NBLK + NW - 1) // NW
NPAD = 10240
ROWS_PER_TILE = NPAD // NS


def _node_tables_body(x_ref, wall_ref, asrc_ref, adst_ref, h_ref, as_ref, ad_ref):
    h = jnp.dot(x_ref[...], wall_ref[...], preferred_element_type=jnp.float32)
    h_ref[...] = h
    as_ref[...] = jnp.dot(h, asrc_ref[...], preferred_element_type=jnp.float32)
    ad_ref[...] = jnp.dot(h, adst_ref[...], preferred_element_type=jnp.float32)


def _node_tables(x, w_all, a_src16, a_dst16):
    bn = 1024
    return pl.pallas_call(
        _node_tables_body,
        grid=(NPAD // bn,),
        in_specs=[
            pl.BlockSpec((bn, D), lambda i: (i, 0)),
            pl.BlockSpec((D, D), lambda i: (0, 0)),
            pl.BlockSpec((D, 16), lambda i: (0, 0)),
            pl.BlockSpec((D, 16), lambda i: (0, 0)),
        ],
        out_specs=[
            pl.BlockSpec((bn, D), lambda i: (i, 0)),
            pl.BlockSpec((bn, 16), lambda i: (i, 0)),
            pl.BlockSpec((bn, 16), lambda i: (i, 0)),
        ],
        out_shape=[
            jax.ShapeDtypeStruct((NPAD, D), jnp.float32),
            jax.ShapeDtypeStruct((NPAD, 16), jnp.float32),
            jax.ShapeDtypeStruct((NPAD, 16), jnp.float32),
        ],
    )(x, w_all, a_src16, a_dst16)


def _edge_table_body(ea_ref, wc_ref, ae_ref):
    ae_ref[...] = jnp.dot(ea_ref[...], wc_ref[...], preferred_element_type=jnp.float32)


def _edge_table(edge_attr, wc16):
    be = 4000
    return pl.pallas_call(
        _edge_table_body,
        grid=(E // be,),
        in_specs=[
            pl.BlockSpec((be, DE), lambda i: (i, 0)),
            pl.BlockSpec((DE, 16), lambda i: (0, 0)),
        ],
        out_specs=pl.BlockSpec((be, 16), lambda i: (i, 0)),
        out_shape=jax.ShapeDtypeStruct((E, 16), jnp.float32),
    )(edge_attr, wc16)


def _edge_kernel_body(src_hbm, dst_hbm, h_hbm, as_hbm, ad_hbm, ae_hbm,
                      znum_hbm, zden_hbm,
                      num0_hbm, num1_hbm, den0_hbm, den1_hbm,
                      src_idx, dst_idx, as_b, ad_b, ae_b, h_b, w_b,
                      num_sh, den_sh, sem0, sem1, sem2):
    c = lax.axis_index("c")
    s = lax.axis_index("s")
    wid = s * NC + c

    r0 = s * ROWS_PER_TILE
    pltpu.sync_copy(znum_hbm.at[pl.ds(r0, ROWS_PER_TILE)],
                    num_sh.at[pl.ds(r0, ROWS_PER_TILE)])
    pltpu.sync_copy(zden_hbm.at[pl.ds(r0, ROWS_PER_TILE)],
                    den_sh.at[pl.ds(r0, ROWS_PER_TILE)])
    plsc.subcore_barrier()

    lanes = lax.iota(jnp.int32, 16)
    headmask = lanes < H

    def step(i, carry):
        b = wid + i * NW

        @pl.when(b < NBLK)
        def _():
            base = b * EB
            pltpu.sync_copy(src_hbm.at[pl.ds(base, EB)], src_idx)
            pltpu.sync_copy(dst_hbm.at[pl.ds(base, EB)], dst_idx)
            pltpu.sync_copy(ae_hbm.at[pl.ds(base, EB)], ae_b)
            pltpu.async_copy(as_hbm.at[src_idx], as_b, sem0).wait()
            pltpu.async_copy(ad_hbm.at[dst_idx], ad_b, sem1).wait()
            pltpu.async_copy(h_hbm.at[src_idx], h_b, sem2).wait()

            @plsc.parallel_loop(0, EB, unroll=2)
            def _(j):
                u = as_b[j, :] + ad_b[j, :] + ae_b[j, :]
                u = jnp.where(u >= 0.0, u, 0.2 * u)
                w = jnp.exp(u)
                w = jnp.where(headmask, w, 0.0)
                w_b[j, :] = w
                for k in range(H):
                    h_b[j, pl.ds(k * HD, HD)] = h_b[j, pl.ds(k * HD, HD)] * w[k]

            pltpu.sync_copy(h_b, num_sh.at[dst_idx], add=True)
            pltpu.sync_copy(w_b, den_sh.at[dst_idx], add=True)

        return carry

    lax.fori_loop(0, STEPS, step, 0)
    plsc.subcore_barrier()

    @pl.when(c == 0)
    def _():
        pltpu.sync_copy(num_sh.at[pl.ds(r0, ROWS_PER_TILE)],
                        num0_hbm.at[pl.ds(r0, ROWS_PER_TILE)])
        pltpu.sync_copy(den_sh.at[pl.ds(r0, ROWS_PER_TILE)],
                        den0_hbm.at[pl.ds(r0, ROWS_PER_TILE)])

    @pl.when(c == 1)
    def _():
        pltpu.sync_copy(num_sh.at[pl.ds(r0, ROWS_PER_TILE)],
                        num1_hbm.at[pl.ds(r0, ROWS_PER_TILE)])
        pltpu.sync_copy(den_sh.at[pl.ds(r0, ROWS_PER_TILE)],
                        den1_hbm.at[pl.ds(r0, ROWS_PER_TILE)])


def _edge_phase(src, dst, h_all, as16, ad16, ae16):
    znum = jnp.zeros((NPAD, D), jnp.float32)
    zden = jnp.zeros((NPAD, 16), jnp.float32)
    run = functools.partial(
        pl.kernel,
        out_type=[
            jax.ShapeDtypeStruct((NPAD, D), jnp.float32),
            jax.ShapeDtypeStruct((NPAD, D), jnp.float32),
            jax.ShapeDtypeStruct((NPAD, 16), jnp.float32),
            jax.ShapeDtypeStruct((NPAD, 16), jnp.float32),
        ],
        mesh=plsc.VectorSubcoreMesh(core_axis_name="c", subcore_axis_name="s"),
        compiler_params=pltpu.CompilerParams(use_tc_tiling_on_sc=False),
        scratch_types=[
            pltpu.VMEM((EB,), jnp.int32),
            pltpu.VMEM((EB,), jnp.int32),
            pltpu.VMEM((EB, 16), jnp.float32),
            pltpu.VMEM((EB, 16), jnp.float32),
            pltpu.VMEM((EB, 16), jnp.float32),
            pltpu.VMEM((EB, D), jnp.float32),
            pltpu.VMEM((EB, 16), jnp.float32),
            pltpu.VMEM_SHARED((NPAD, D), jnp.float32),
            pltpu.VMEM_SHARED((NPAD, 16), jnp.float32),
            pltpu.SemaphoreType.DMA,
            pltpu.SemaphoreType.DMA,
            pltpu.SemaphoreType.DMA,
        ],
    )(_edge_kernel_body)
    return run(src, dst, h_all, as16, ad16, ae16, znum, zden)


def _leaky(v, s):
    return jnp.where(v >= 0.0, v, s * v)


def _epilogue_body(num0_ref, num1_ref, den0_ref, den1_ref, rep_ref,
                   bconv_ref, wfc_ref, bfc_ref, lng_ref, lnb_ref,
                   wgate_ref, bgate_ref, wglob_ref, bglob_ref, out_ref):
    den = den0_ref[pl.ds(0, N), :] + den1_ref[pl.ds(0, N), :]
    den_rep = jnp.dot(den, rep_ref[...], preferred_element_type=jnp.float32)
    x = ((num0_ref[pl.ds(0, N), :] + num1_ref[pl.ds(0, N), :])
         / (den_rep + 1e-16) + bconv_ref[...])
    t = jnp.dot(x, wfc_ref[...], preferred_element_type=jnp.float32) + bfc_ref[...]
    t = _leaky(t, 0.01)
    t = t - jnp.max(t, axis=-1, keepdims=True)
    et = jnp.exp(t)
    sa = et / jnp.sum(et, axis=-1, keepdims=True)
    x = _leaky(x * sa, 0.2)
    x = jnp.dot(x, wfc_ref[...], preferred_element_type=jnp.float32) + bfc_ref[...]
    mu = jnp.mean(x, axis=-1, keepdims=True)
    xc = x - mu
    var = jnp.mean(xc * xc, axis=-1, keepdims=True)
    x = xc * jax.lax.rsqrt(var + 1e-5) * lng_ref[...] + lnb_ref[...]
    nrm = jnp.sqrt(jnp.sum(x * x, axis=-1, keepdims=True))
    x = x / jnp.maximum(nrm, 1e-12)
    g = jnp.sum(x * wgate_ref[...], axis=-1, keepdims=True) + bgate_ref[0, 0]
    g = g - jnp.max(g)
    eg = jnp.exp(g)
    gate = eg / jnp.sum(eg)
    xg = jnp.sum(gate * x, axis=0, keepdims=True)
    q = jnp.dot(xg, wglob_ref[...], preferred_element_type=jnp.float32) + bglob_ref[...]
    q = jnp.maximum(q, 0.0)
    q = q - jnp.max(q, axis=-1, keepdims=True)
    eq = jnp.exp(q)
    ga = eq / jnp.sum(eq, axis=-1, keepdims=True)
    out_ref[...] = x * ga


def _epilogue(num0, num1, den0, den1, rep, bconv_row, wfc, bfc_row, lng_row,
              lnb_row, wgate_row, bgate2, wglob, bglob_row):
    return pl.pallas_call(
        _epilogue_body,
        out_shape=jax.ShapeDtypeStruct((N, D), jnp.float32),
    )(num0, num1, den0, den1, rep, bconv_row, wfc, bfc_row, lng_row,
      lnb_row, wgate_row, bgate2, wglob, bglob_row)


def kernel(x, edge_index, edge_attr, W, att_src, att_dst, W_edge, att_edge,
           b_conv, W_fc, b_fc, ln_g, ln_b, W_gate, b_gate, W_glob, b_glob):
    src = edge_index[0].astype(jnp.int32)
    dst = edge_index[1].astype(jnp.int32)

    w_all = jnp.transpose(W, (1, 0, 2)).reshape(D, H * HD)
    eye = jnp.eye(H, dtype=jnp.float32)
    a_src16 = jnp.concatenate(
        [(eye[:, None, :] * att_src[:, :, None]).reshape(H * HD, H),
         jnp.zeros((H * HD, 16 - H), jnp.float32)], axis=1)
    a_dst16 = jnp.concatenate(
        [(eye[:, None, :] * att_dst[:, :, None]).reshape(H * HD, H),
         jnp.zeros((H * HD, 16 - H), jnp.float32)], axis=1)
    wc16 = jnp.concatenate(
        [jnp.einsum("hdk,hk->dh", W_edge, att_edge),
         jnp.zeros((DE, 16 - H), jnp.float32)], axis=1)
    rep = jnp.concatenate(
        [jnp.repeat(jnp.eye(H, dtype=jnp.float32), HD, axis=1),
         jnp.zeros((16 - H, D), jnp.float32)], axis=0)

    xp = jnp.concatenate([x, jnp.zeros((NPAD - N, D), jnp.float32)], axis=0)
    h_all, as16, ad16 = _node_tables(xp, w_all, a_src16, a_dst16)
    ae16 = _edge_table(edge_attr, wc16)
    num0, num1, den0, den1 = _edge_phase(src, dst, h_all, as16, ad16, ae16)

    return _epilogue(
        num0, num1, den0, den1, rep,
        b_conv.reshape(1, D), W_fc, b_fc.reshape(1, D), ln_g.reshape(1, D),
        ln_b.reshape(1, D), W_gate.reshape(1, D), b_gate.reshape(1, 1),
        W_glob, b_glob.reshape(1, D))
